# baseline (speedup 1.0000x reference)
import functools

import jax
import jax.numpy as jnp
from jax import lax
from jax.experimental import pallas as pl
from jax.experimental.pallas import tpu as pltpu
from jax.experimental.pallas import tpu_sc as plsc

B = 16384
VOCAB = 1000000
DIM = 128
WALK = 5

NC = 2
NS = 16
NW = NC * NS
BPW = B // NW
CHUNK = 128
NCHUNK = BPW // CHUNK
L = 16
KD = DIM // L


def _fix_idx(v):
    return jnp.where(v == 0, VOCAB - 1, v - 1)


def _sc_partials(s, w, X):
    mesh = plsc.VectorSubcoreMesh(
        core_axis_name="c", subcore_axis_name="s", num_cores=NC, num_subcores=NS
    )
    out_sds = jax.ShapeDtypeStruct((3 * NW, L), jnp.float32)

    scratch = [
        pltpu.VMEM((BPW,), jnp.int32),
        pltpu.VMEM((NCHUNK, CHUNK), jnp.int32),
        pltpu.VMEM((WALK, NCHUNK, CHUNK), jnp.int32),
    ] + [
        pltpu.VMEM((CHUNK, DIM), jnp.float32) for _ in range(WALK + 1)
    ] + [
        pltpu.VMEM((BPW,), jnp.float32) for _ in range(WALK)
    ] + [
        pltpu.VMEM((L,), jnp.float32),
        pltpu.SemaphoreType.DMA,
    ]

    @functools.partial(
        pl.kernel,
        out_type=out_sds,
        mesh=mesh,
        scratch_types=scratch,
        compiler_params=pltpu.CompilerParams(needs_layout_passes=False),
    )
    def sc_k(s_hbm, wt_hbm, x_hbm, out_hbm, *refs):
        raw = refs[0]
        idx_s = refs[1]
        idx_w = refs[2]
        bufs = refs[3:3 + (WALK + 1)]
        scores = refs[3 + (WALK + 1):3 + (WALK + 1) + WALK]
        stage = refs[3 + (WALK + 1) + WALK]
        sem = refs[4 + (WALK + 1) + WALK]

        wid = lax.axis_index("c") * NS + lax.axis_index("s")
        base = wid * BPW
        lanes = lax.iota(jnp.int32, L)

        pltpu.sync_copy(s_hbm.at[pl.ds(base, BPW)], raw)
        for c in range(NCHUNK):
            for t in range(CHUNK // L):
                v = raw[pl.ds(c * CHUNK + t * L, L)]
                idx_s[c, pl.ds(t * L, L)] = _fix_idx(v)
        for j in range(WALK):
            pltpu.sync_copy(wt_hbm.at[pl.ds(j * B + base, BPW)], raw)
            for c in range(NCHUNK):
                for t in range(CHUNK // L):
                    v = raw[pl.ds(c * CHUNK + t * L, L)]
                    idx_w[j, c, pl.ds(t * L, L)] = _fix_idx(v)

        def chunk_body(c, _):
            cps = [pltpu.async_copy(x_hbm.at[idx_s.at[c]], bufs[0], sem)]
            for j in range(WALK):
                cps.append(
                    pltpu.async_copy(x_hbm.at[idx_w.at[j, c]], bufs[1 + j], sem)
                )
            for cp in cps:
                cp.wait()

            def dot_body(g, _):
                vecs = [jnp.zeros((L,), jnp.float32) for _ in range(WALK)]
                for bi in range(L):
                    b = g * L + bi
                    sv = [bufs[0][b, pl.ds(k * L, L)] for k in range(KD)]
                    for j in range(WALK):
                        wr = bufs[1 + j]
                        acc = sv[0] * wr[b, pl.ds(0, L)]
                        for k in range(1, KD):
                            acc = acc + sv[k] * wr[b, pl.ds(k * L, L)]
                        vecs[j] = jnp.where(lanes == bi, jnp.sum(acc), vecs[j])
                for j in range(WALK):
                    scores[j][pl.ds(c * CHUNK + g * L, L)] = vecs[j]
                return 0

            lax.fori_loop(0, CHUNK // L, dot_body, 0)
            return 0

        lax.fori_loop(0, NCHUNK, chunk_body, 0)

        m_vec = jnp.zeros((L,), jnp.float32)
        se_vec = jnp.zeros((L,), jnp.float32)
        ss_vec = jnp.zeros((L,), jnp.float32)
        for j in range(WALK):
            def max_body(t, m, j=j):
                return jnp.maximum(m, jnp.max(scores[j][pl.ds(t * L, L)]))

            m = lax.fori_loop(0, BPW // L, max_body, jnp.float32(-3e38))

            def sum_body(t, carry, j=j, m=m):
                se, ss = carry
                v = scores[j][pl.ds(t * L, L)]
                return se + jnp.exp(v - m), ss + v

            zero = jnp.zeros((L,), jnp.float32)
            se_l, ss_l = lax.fori_loop(0, BPW // L, sum_body, (zero, zero))
            m_vec = jnp.where(lanes == j, m, m_vec)
            se_vec = jnp.where(lanes == j, jnp.sum(se_l), se_vec)
            ss_vec = jnp.where(lanes == j, jnp.sum(ss_l), ss_vec)

        stage[...] = m_vec
        pltpu.sync_copy(stage, out_hbm.at[wid])
        stage[...] = se_vec
        pltpu.sync_copy(stage, out_hbm.at[NW + wid])
        stage[...] = ss_vec
        pltpu.sync_copy(stage, out_hbm.at[2 * NW + wid])

    return sc_k(s, w, X)


def _tc_finalize(p):
    def tc_body(p_ref, o_ref):
        pv = p_ref[...]
        mv = pv[:NW, :]
        sev = pv[NW:2 * NW, :]
        ssv = pv[2 * NW:, :]
        mx = jnp.max(mv, axis=0, keepdims=True)
        se_tot = jnp.sum(sev * jnp.exp(mv - mx), axis=0, keepdims=True)
        ss_tot = jnp.sum(ssv, axis=0, keepdims=True)
        res = B * (mx + jnp.log(se_tot)) - ss_tot
        o_ref[...] = res[:, :8]

    return pl.pallas_call(
        tc_body,
        out_shape=jax.ShapeDtypeStruct((1, 8), jnp.float32),
    )(p)


def kernel(s, w, neg, X):
    del neg
    wt = jnp.transpose(w.astype(jnp.int32)).reshape(-1)
    p = _sc_partials(s.astype(jnp.int32), wt, X)
    return _tc_finalize(p)[0, :WALK]


# speedup vs baseline: 1.0218x; 1.0218x over previous
import functools

import jax
import jax.numpy as jnp
from jax import lax
from jax.experimental import pallas as pl
from jax.experimental.pallas import tpu as pltpu
from jax.experimental.pallas import tpu_sc as plsc

B = 16384
VOCAB = 1000000
DIM = 128
WALK = 5

NC = 2
NS = 16
NW = NC * NS
BPW = B // NW
CHUNK = 128
NCHUNK = BPW // CHUNK
L = 16
KD = DIM // L


def _fix_idx(v):
    return jnp.where(v == 0, VOCAB - 1, v - 1)


def _sc_partials(s, w, X):
    mesh = plsc.VectorSubcoreMesh(
        core_axis_name="c", subcore_axis_name="s", num_cores=NC, num_subcores=NS
    )
    out_sds = jax.ShapeDtypeStruct((3 * NW, L), jnp.float32)

    scratch = [
        pltpu.VMEM((BPW,), jnp.int32),
        pltpu.VMEM((NCHUNK, CHUNK), jnp.int32),
        pltpu.VMEM((WALK, NCHUNK, CHUNK), jnp.int32),
    ] + [
        pltpu.VMEM((CHUNK, DIM), jnp.float32) for _ in range(WALK + 1)
    ] + [
        pltpu.VMEM((BPW,), jnp.float32) for _ in range(WALK)
    ] + [
        pltpu.VMEM((L,), jnp.float32),
        pltpu.SemaphoreType.DMA,
    ]

    @functools.partial(
        pl.kernel,
        out_type=out_sds,
        mesh=mesh,
        scratch_types=scratch,
        compiler_params=pltpu.CompilerParams(needs_layout_passes=False),
    )
    def sc_k(s_hbm, wt_hbm, x_hbm, out_hbm, *refs):
        raw = refs[0]
        idx_s = refs[1]
        idx_w = refs[2]
        bufs = refs[3:3 + (WALK + 1)]
        scores = refs[3 + (WALK + 1):3 + (WALK + 1) + WALK]
        stage = refs[3 + (WALK + 1) + WALK]
        sem = refs[4 + (WALK + 1) + WALK]

        wid = lax.axis_index("c") * NS + lax.axis_index("s")
        base = wid * BPW
        lanes = lax.iota(jnp.int32, L)

        pltpu.sync_copy(s_hbm.at[pl.ds(base, BPW)], raw)
        for c in range(NCHUNK):
            for t in range(CHUNK // L):
                v = raw[pl.ds(c * CHUNK + t * L, L)]
                idx_s[c, pl.ds(t * L, L)] = _fix_idx(v)
        for j in range(WALK):
            pltpu.sync_copy(wt_hbm.at[pl.ds(j * B + base, BPW)], raw)
            for c in range(NCHUNK):
                for t in range(CHUNK // L):
                    v = raw[pl.ds(c * CHUNK + t * L, L)]
                    idx_w[j, c, pl.ds(t * L, L)] = _fix_idx(v)

        def chunk_body(c, _):
            cps = [pltpu.async_copy(x_hbm.at[idx_s.at[c]], bufs[0], sem)]
            for j in range(WALK):
                cps.append(
                    pltpu.async_copy(x_hbm.at[idx_w.at[j, c]], bufs[1 + j], sem)
                )
            for cp in cps:
                cp.wait()

            @plsc.parallel_loop(0, CHUNK // L)
            def dot_body(g):
                vecs = [jnp.zeros((L,), jnp.float32) for _ in range(WALK)]
                for bi in range(L):
                    b = g * L + bi
                    sv = [bufs[0][b, pl.ds(k * L, L)] for k in range(KD)]
                    for j in range(WALK):
                        wr = bufs[1 + j]
                        acc = sv[0] * wr[b, pl.ds(0, L)]
                        for k in range(1, KD):
                            acc = acc + sv[k] * wr[b, pl.ds(k * L, L)]
                        vecs[j] = jnp.where(lanes == bi, jnp.sum(acc), vecs[j])
                for j in range(WALK):
                    scores[j][pl.ds(c * CHUNK + g * L, L)] = vecs[j]

            return 0

        lax.fori_loop(0, NCHUNK, chunk_body, 0)

        m_vec = jnp.zeros((L,), jnp.float32)
        se_vec = jnp.zeros((L,), jnp.float32)
        ss_vec = jnp.zeros((L,), jnp.float32)
        for j in range(WALK):
            def max_body(t, m, j=j):
                return jnp.maximum(m, jnp.max(scores[j][pl.ds(t * L, L)]))

            m = lax.fori_loop(0, BPW // L, max_body, jnp.float32(-3e38))

            def sum_body(t, carry, j=j, m=m):
                se, ss = carry
                v = scores[j][pl.ds(t * L, L)]
                return se + jnp.exp(v - m), ss + v

            zero = jnp.zeros((L,), jnp.float32)
            se_l, ss_l = lax.fori_loop(0, BPW // L, sum_body, (zero, zero))
            m_vec = jnp.where(lanes == j, m, m_vec)
            se_vec = jnp.where(lanes == j, jnp.sum(se_l), se_vec)
            ss_vec = jnp.where(lanes == j, jnp.sum(ss_l), ss_vec)

        stage[...] = m_vec
        pltpu.sync_copy(stage, out_hbm.at[wid])
        stage[...] = se_vec
        pltpu.sync_copy(stage, out_hbm.at[NW + wid])
        stage[...] = ss_vec
        pltpu.sync_copy(stage, out_hbm.at[2 * NW + wid])

    return sc_k(s, w, X)


def _tc_finalize(p):
    def tc_body(p_ref, o_ref):
        pv = p_ref[...]
        mv = pv[:NW, :]
        sev = pv[NW:2 * NW, :]
        ssv = pv[2 * NW:, :]
        mx = jnp.max(mv, axis=0, keepdims=True)
        se_tot = jnp.sum(sev * jnp.exp(mv - mx), axis=0, keepdims=True)
        ss_tot = jnp.sum(ssv, axis=0, keepdims=True)
        res = B * (mx + jnp.log(se_tot)) - ss_tot
        o_ref[...] = res[:, :8]

    return pl.pallas_call(
        tc_body,
        out_shape=jax.ShapeDtypeStruct((1, 8), jnp.float32),
    )(p)


def kernel(s, w, neg, X):
    del neg
    wt = jnp.transpose(w.astype(jnp.int32)).reshape(-1)
    p = _sc_partials(s.astype(jnp.int32), wt, X)
    return _tc_finalize(p)[0, :WALK]
